# baseline (speedup 1.0000x reference)
import jax
import jax.numpy as jnp
from jax.experimental import pallas as pl
from jax.experimental.pallas import tpu as pltpu

B, N, F = 32, 512, 128
GCN_UNITS = 32
DENSE_UNITS = 512
DEPTH = 16
NROW = F + 1


def _xw_from_view(x2, w16):
    lane = jax.lax.broadcasted_iota(jnp.int32, (F, F), 1)
    row = jax.lax.broadcasted_iota(jnp.int32, (F, F), 0)
    sel = lane <= row
    jrev = (row + lane == F - 1).astype(jnp.bfloat16)
    parts = []
    for q in range(N // F):
        a_rows = x2[NROW * q:NROW * q + F, :].astype(jnp.bfloat16)
        b_rows = x2[NROW * q + 1:NROW * q + 1 + F, :].astype(jnp.bfloat16)
        a_f = jnp.dot(jrev, a_rows, preferred_element_type=jnp.float32)
        b_f = jnp.dot(jrev, b_rows, preferred_element_type=jnp.float32)
        ra = pltpu.roll(a_f, 1, 1, stride=1, stride_axis=0)
        rb = pltpu.roll(b_f, 1, 1, stride=1, stride_axis=0)
        blk_rev = jnp.where(sel, ra, rb)
        xw_rev = jnp.dot(blk_rev.astype(jnp.bfloat16), w16,
                         preferred_element_type=jnp.float32)
        parts.append(jnp.dot(jrev, xw_rev.astype(jnp.bfloat16),
                             preferred_element_type=jnp.float32))
    return jnp.concatenate(parts, axis=0)


def _fused_kernel(x2_ref, a_hbm, wg_ref, bg_ref, w1_ref, b1_ref, w2_ref,
                  b2_ref, out_ref, abuf, asem, pooled):
    b = pl.program_id(0)

    @pl.when(b == 0)
    def _prologue():
        for d in range(DEPTH):
            pltpu.make_async_copy(a_hbm.at[d], abuf.at[d], asem.at[d]).start()

    slot = jax.lax.rem(b, DEPTH)
    pltpu.make_async_copy(a_hbm.at[b], abuf.at[slot], asem.at[slot]).wait()

    xw = _xw_from_view(x2_ref[0], wg_ref[:, :].astype(jnp.bfloat16))
    h = jnp.dot(abuf[slot].astype(jnp.bfloat16), xw.astype(jnp.bfloat16),
                preferred_element_type=jnp.float32)
    bg = bg_ref[0, :]
    pooled[b, :GCN_UNITS] = jnp.mean(h, axis=0) + bg
    pooled[b, GCN_UNITS:] = jnp.max(h, axis=0) + bg

    @pl.when(b + DEPTH < B)
    def _next():
        pltpu.make_async_copy(a_hbm.at[b + DEPTH], abuf.at[slot],
                              asem.at[slot]).start()

    @pl.when(b == B - 1)
    def _head():
        p = pooled[:, :]
        z = jnp.dot(p, w1_ref[:, :], preferred_element_type=jnp.float32)
        z = jnp.maximum(z + b1_ref[0, :], 0.0)
        out = jnp.dot(z, w2_ref[:, :], preferred_element_type=jnp.float32)
        out_ref[:, :] = out + b2_ref[0, :]


@jax.jit
def kernel(x, a, W_gcn, b_gcn, W1, b1, W2, b2):
    x2 = x.reshape(B, (N * NROW) // F, F)
    return pl.pallas_call(
        _fused_kernel,
        grid=(B,),
        in_specs=[
            pl.BlockSpec((1, (N * NROW) // F, F), lambda b: (b, 0, 0)),
            pl.BlockSpec(memory_space=pl.ANY),
            pl.BlockSpec((F, GCN_UNITS), lambda b: (0, 0)),
            pl.BlockSpec((1, GCN_UNITS), lambda b: (0, 0)),
            pl.BlockSpec((2 * GCN_UNITS, DENSE_UNITS), lambda b: (0, 0)),
            pl.BlockSpec((1, DENSE_UNITS), lambda b: (0, 0)),
            pl.BlockSpec((DENSE_UNITS, 1), lambda b: (0, 0)),
            pl.BlockSpec((1, 1), lambda b: (0, 0)),
        ],
        out_specs=pl.BlockSpec((B, 1), lambda b: (0, 0)),
        out_shape=jax.ShapeDtypeStruct((B, 1), jnp.float32),
        scratch_shapes=[
            pltpu.VMEM((DEPTH, N, N), jnp.float32),
            pltpu.SemaphoreType.DMA((DEPTH,)),
            pltpu.VMEM((B, 2 * GCN_UNITS), jnp.float32),
        ],
    )(x2, a, W_gcn, b_gcn.reshape(1, GCN_UNITS), W1,
      b1.reshape(1, DENSE_UNITS), W2, b2.reshape(1, 1))
